# Initial kernel scaffold; baseline (speedup 1.0000x reference)
import jax
import jax.numpy as jnp
from jax.experimental import pallas as pl


def kernel(token_embeddings, tokens2edges, edge_index, edges2tokens, W, b):
    raise NotImplementedError("write your pallas kernel here")


# speedup vs baseline: 11.2880x; 11.2880x over previous
import functools

import jax
import jax.numpy as jnp
from jax import lax
from jax.experimental import pallas as pl
from jax.experimental.pallas import tpu as pltpu
from jax.experimental.pallas import tpu_sc as plsc

L = 16
NT = 16
NC = 2
CSE = 96
CSN = 80
NB = 8
EB = 1024


def _rsqrt16(x):
    i = lax.bitcast_convert_type(x, jnp.int32)
    i = jnp.int32(0x5F3759DF) - (i >> 1)
    y = lax.bitcast_convert_type(i, jnp.float32)
    for _ in range(3):
        y = y * (1.5 - 0.5 * x * y * y)
    return y


def _matmul(x_flat, W):
    n, d = x_flat.shape
    blk = 1024

    def body(x_ref, w_ref, o_ref):
        o_ref[...] = jnp.dot(x_ref[...], w_ref[...],
                             preferred_element_type=jnp.float32)

    return pl.pallas_call(
        body,
        grid=(n // blk,),
        in_specs=[pl.BlockSpec((blk, d), lambda i: (i, 0)),
                  pl.BlockSpec((d, d), lambda i: (0, 0))],
        out_specs=pl.BlockSpec((blk, d), lambda i: (i, 0)),
        out_shape=jax.ShapeDtypeStruct((n, d), jnp.float32),
    )(x_flat, W)


def kernel(token_embeddings, tokens2edges, edge_index, edges2tokens, W, b):
    B, T, D = token_embeddings.shape
    E = edge_index.shape[2]
    TPAD = ((T + NT * CSN - 1) // (NT * CSN)) * (NT * CSN)
    ST = TPAD + 8
    ZROW = TPAD
    NCHN = TPAD // CSN
    NOUTCH = T // CSN
    steps = -(-E // (NT * CSE))
    steps += -steps % NB
    nblk = steps // NB
    pad_e = NT * steps * CSE - E

    x_flat = jnp.pad(token_embeddings, ((0, 0), (0, TPAD - T), (0, 0))
                     ).reshape(B * TPAD, D)
    boffs_st = (jnp.arange(B, dtype=jnp.int32) * ST)[:, None]
    boffs_tp = (jnp.arange(B, dtype=jnp.int32) * TPAD)[:, None]
    src_g = (jnp.pad(edge_index[:, 0, :], ((0, 0), (0, pad_e))) + boffs_st
             ).reshape(B * NT, steps, CSE)
    dst_g = jnp.pad(edge_index[:, 1, :], ((0, 0), (0, pad_e)),
                    constant_values=T).reshape(B * NT, steps, CSE)
    t2e_g = (jnp.pad(tokens2edges, ((0, 0), (0, TPAD - T))) + boffs_tp
             ).reshape(B * TPAD)
    e2t_flat = jnp.pad(edges2tokens, ((0, 0), (0, TPAD - T)),
                       constant_values=T).reshape(B * TPAD)

    H_flat = _matmul(x_flat, W)

    def body(H, x, srcg, dstg, t2eg, e2t, bvec, out, work,
             sidx, didx, rows_a, rows_b, idx_buf, m_buf, deg_buf,
             dis_t, ones_buf, bias_buf, zdeg, e2t_blk, m_v, tmp16,
             acc_sh, deg_sh, m_sh, sem_a, sem_b):
        ci = lax.axis_index("c")
        s = lax.axis_index("s")
        lane = lax.iota(jnp.int32, L)
        tile_src = srcg.at[ci * NT + s]
        tile_dst = dstg.at[ci * NT + s]

        pltpu.sync_copy(bvec, bias_buf)
        zf = jnp.zeros((L,), jnp.float32)

        @pl.loop(0, CSE * D // L)
        def _(t):
            rows_a[t >> 3, pl.ds((t & 7) * L, L)] = zf

        @pl.loop(0, TPAD // NT // L)
        def _(t):
            zdeg[pl.ds(t * L, L)] = zf

        @pl.loop(0, CSE // L)
        def _(t):
            ones_buf[pl.ds(t * L, L)] = jnp.ones((L,), jnp.float32)

        @pl.loop(0, TPAD // NT // CSN)
        def _(i):
            pltpu.sync_copy(rows_a.at[pl.ds(0, CSN)],
                            acc_sh.at[pl.ds(s * (TPAD // NT) + i * CSN, CSN)])

        pltpu.sync_copy(zdeg, deg_sh.at[pl.ds(s * (TPAD // NT), TPAD // NT)])

        @pl.when(s == 0)
        def _():
            pltpu.sync_copy(rows_a.at[pl.ds(0, 8)],
                            work.at[pl.ds(ci * ST + ZROW, 8)])

        plsc.subcore_barrier()

        @pl.when(s == 0)
        def _():
            zi = jnp.zeros((L,), jnp.int32)

            @pl.loop(0, TPAD // L)
            def _(i):
                m_v[pl.ds(i * L, L)] = zi

            @pl.loop(0, TPAD // EB)
            def _(cb):
                pltpu.sync_copy(e2t.at[pl.ds(ci * TPAD + cb * EB, EB)],
                                e2t_blk)

                @pl.loop(0, EB // L)
                def _(jj):
                    ev = e2t_blk[pl.ds(jj * L, L)]
                    tmp16[...] = ev
                    dup = lane < 0
                    for shift in range(1, L):
                        rot = plsc.load_gather(
                            tmp16, [jnp.minimum(lane + shift, L - 1)])
                        dup = dup | ((rot == ev) & (lane < L - shift))
                    plsc.store_scatter(m_v, [ev],
                                       cb * EB + jj * L + lane + 1,
                                       mask=jnp.logical_not(dup))

            pltpu.sync_copy(m_v, m_sh)

        @pl.loop(0, nblk)
        def _(blk):
            pltpu.sync_copy(tile_dst.at[pl.ds(blk * NB, NB)], didx)

            @pl.loop(0, NB)
            def _(st):
                pltpu.sync_copy(ones_buf, deg_sh.at[didx.at[st]], add=True)

        plsc.subcore_barrier()

        for k in range(NCHN // NT):
            c = s * (NCHN // NT) + k

            @pl.when(c < NOUTCH)
            def _():
                base = c * CSN
                pltpu.sync_copy(deg_sh.at[pl.ds(base, CSN)], deg_buf)

                @pl.loop(0, CSN // L)
                def _(i):
                    d = deg_buf[pl.ds(i * L, L)] + 1.0
                    dis_t[pl.ds(k * CSN + i * L, L)] = _rsqrt16(d)

                pltpu.sync_copy(t2eg.at[pl.ds(ci * TPAD + base, CSN)],
                                idx_buf)
                pltpu.async_copy(H.at[idx_buf], rows_a.at[pl.ds(0, CSN)],
                                 sem_a).wait()

                @pl.loop(0, CSN)
                def _(r):
                    sv = plsc.load_gather(
                        dis_t, [jnp.full((L,), k * CSN + r, dtype=jnp.int32)])
                    for q in range(D // L):
                        sl = pl.ds(q * L, L)
                        rows_a[r, sl] = rows_a[r, sl] * sv

                pltpu.sync_copy(rows_a.at[pl.ds(0, CSN)],
                                work.at[pl.ds(ci * ST + base, CSN)])

        plsc.subcore_barrier()

        @pl.loop(0, nblk)
        def _(blk):
            pltpu.sync_copy(tile_src.at[pl.ds(blk * NB, NB)], sidx)
            pltpu.sync_copy(tile_dst.at[pl.ds(blk * NB, NB)], didx)

            @pl.loop(0, NB, step=2)
            def _(st):
                d_a = pltpu.async_copy(work.at[sidx.at[st]], rows_a, sem_a)
                d_b = pltpu.async_copy(work.at[sidx.at[st + 1]], rows_b,
                                       sem_b)
                d_a.wait()
                pltpu.sync_copy(rows_a, acc_sh.at[didx.at[st]], add=True)
                d_b.wait()
                pltpu.sync_copy(rows_b, acc_sh.at[didx.at[st + 1]], add=True)

        plsc.subcore_barrier()

        for k in range(NCHN // NT):
            c = s * (NCHN // NT) + k

            @pl.when(c < NOUTCH)
            def _():
                base = c * CSN
                pltpu.sync_copy(acc_sh.at[pl.ds(base, CSN)],
                                rows_a.at[pl.ds(0, CSN)])
                pltpu.sync_copy(work.at[pl.ds(ci * ST + base, CSN)],
                                rows_b.at[pl.ds(0, CSN)])

                @pl.loop(0, CSN)
                def _(r):
                    sv = plsc.load_gather(
                        dis_t, [jnp.full((L,), k * CSN + r, dtype=jnp.int32)])
                    for q in range(D // L):
                        sl = pl.ds(q * L, L)
                        rows_a[r, sl] = ((rows_a[r, sl] + rows_b[r, sl]) * sv
                                         + bias_buf[sl])

                pltpu.sync_copy(rows_a.at[pl.ds(0, CSN)],
                                work.at[pl.ds(ci * ST + base, CSN)])

        plsc.subcore_barrier()

        for k in range(NCHN // NT):
            c = s * (NCHN // NT) + k

            @pl.when(c < NOUTCH)
            def _():
                base = c * CSN
                pltpu.sync_copy(m_sh.at[pl.ds(base, CSN)], m_buf)
                off = ci * ST

                @pl.loop(0, CSN // L)
                def _(i):
                    mv = m_buf[pl.ds(i * L, L)]
                    idx_buf[pl.ds(i * L, L)] = (
                        jnp.where(mv >= 2, mv - 2, ZROW) + off)

                pltpu.async_copy(work.at[idx_buf], rows_a.at[pl.ds(0, CSN)],
                                 sem_a).wait()
                pltpu.sync_copy(x.at[pl.ds(ci * TPAD + base, CSN)],
                                rows_b.at[pl.ds(0, CSN)])

                @pl.loop(0, CSN)
                def _(r):
                    for q in range(D // L):
                        sl = pl.ds(q * L, L)
                        rows_a[r, sl] = rows_a[r, sl] + rows_b[r, sl]

                pltpu.sync_copy(rows_a.at[pl.ds(0, CSN)],
                                out.at[pl.ds(ci * T + base, CSN)])

    mesh = plsc.VectorSubcoreMesh(core_axis_name="c", subcore_axis_name="s")
    out_flat, _ = pl.kernel(
        body,
        out_type=[jax.ShapeDtypeStruct((B * T, D), jnp.float32),
                  jax.ShapeDtypeStruct((B * ST, D), jnp.float32)],
        mesh=mesh,
        compiler_params=pltpu.CompilerParams(needs_layout_passes=False),
        scratch_types=[
            pltpu.VMEM((NB, CSE), jnp.int32),
            pltpu.VMEM((NB, CSE), jnp.int32),
            pltpu.VMEM((CSE, D), jnp.float32),
            pltpu.VMEM((CSE, D), jnp.float32),
            pltpu.VMEM((CSN,), jnp.int32),
            pltpu.VMEM((CSN,), jnp.int32),
            pltpu.VMEM((CSN,), jnp.float32),
            pltpu.VMEM((8 * CSN,), jnp.float32),
            pltpu.VMEM((CSE,), jnp.float32),
            pltpu.VMEM((D,), jnp.float32),
            pltpu.VMEM((TPAD // NT,), jnp.float32),
            pltpu.VMEM((EB,), jnp.int32),
            pltpu.VMEM((TPAD,), jnp.int32),
            pltpu.VMEM((L,), jnp.int32),
            pltpu.VMEM_SHARED((TPAD, D), jnp.float32),
            pltpu.VMEM_SHARED((TPAD,), jnp.float32),
            pltpu.VMEM_SHARED((TPAD,), jnp.int32),
            pltpu.SemaphoreType.DMA,
            pltpu.SemaphoreType.DMA,
        ],
    )(H_flat, x_flat, src_g, dst_g, t2e_g, e2t_flat, b)
    return out_flat.reshape(B, T, D)


# speedup vs baseline: 12.0795x; 1.0701x over previous
import functools

import jax
import jax.numpy as jnp
from jax import lax
from jax.experimental import pallas as pl
from jax.experimental.pallas import tpu as pltpu
from jax.experimental.pallas import tpu_sc as plsc

L = 16
NT = 16
NC = 2
CSE = 128
CSN = 80
NB = 8
EB = 1024


def _rsqrt16(x):
    i = lax.bitcast_convert_type(x, jnp.int32)
    i = jnp.int32(0x5F3759DF) - (i >> 1)
    y = lax.bitcast_convert_type(i, jnp.float32)
    for _ in range(3):
        y = y * (1.5 - 0.5 * x * y * y)
    return y


def _matmul(x_flat, W):
    n, d = x_flat.shape
    blk = 1024

    def body(x_ref, w_ref, o_ref):
        o_ref[...] = jnp.dot(x_ref[...], w_ref[...],
                             preferred_element_type=jnp.float32)

    return pl.pallas_call(
        body,
        grid=(n // blk,),
        in_specs=[pl.BlockSpec((blk, d), lambda i: (i, 0)),
                  pl.BlockSpec((d, d), lambda i: (0, 0))],
        out_specs=pl.BlockSpec((blk, d), lambda i: (i, 0)),
        out_shape=jax.ShapeDtypeStruct((n, d), jnp.float32),
    )(x_flat, W)


def kernel(token_embeddings, tokens2edges, edge_index, edges2tokens, W, b):
    B, T, D = token_embeddings.shape
    E = edge_index.shape[2]
    TPAD = ((T + NT * CSN - 1) // (NT * CSN)) * (NT * CSN)
    ST = TPAD + 8
    ZROW = TPAD
    NCHN = TPAD // CSN
    NOUTCH = T // CSN
    steps = -(-E // (NT * CSE))
    steps += -steps % NB
    nblk = steps // NB
    pad_e = NT * steps * CSE - E

    x_flat = jnp.pad(token_embeddings, ((0, 0), (0, TPAD - T), (0, 0))
                     ).reshape(B * TPAD, D)
    boffs_st = (jnp.arange(B, dtype=jnp.int32) * ST)[:, None]
    boffs_tp = (jnp.arange(B, dtype=jnp.int32) * TPAD)[:, None]
    src_g = (jnp.pad(edge_index[:, 0, :], ((0, 0), (0, pad_e))) + boffs_st
             ).reshape(B * NT, steps, CSE)
    dst_g = jnp.pad(edge_index[:, 1, :], ((0, 0), (0, pad_e)),
                    constant_values=T).reshape(B * NT, steps, CSE)
    t2e_g = (jnp.pad(tokens2edges, ((0, 0), (0, TPAD - T))) + boffs_tp
             ).reshape(B * TPAD)
    e2t_2d = jnp.pad(edges2tokens, ((0, 0), (0, TPAD - T)),
                     constant_values=T).reshape(B * TPAD // CSE, CSE)

    H_flat = _matmul(x_flat, W)

    def body(H, x, srcg, dstg, t2eg, e2t, bvec, out, work,
             sidx, didx, rows_a, rows_b, idx_buf, m_buf, deg_buf,
             dis_t, ones_buf, bias_buf, zdeg, zm, vals_buf,
             acc_sh, deg_sh, m_sh, sem_a, sem_b):
        ci = lax.axis_index("c")
        s = lax.axis_index("s")
        lane = lax.iota(jnp.int32, L)
        tile_src = srcg.at[ci * NT + s]
        tile_dst = dstg.at[ci * NT + s]

        pltpu.sync_copy(bvec, bias_buf)
        zf = jnp.zeros((L,), jnp.float32)

        @pl.loop(0, CSE * D // L)
        def _(t):
            rows_a[t >> 3, pl.ds((t & 7) * L, L)] = zf

        @pl.loop(0, TPAD // NT // L)
        def _(t):
            zdeg[pl.ds(t * L, L)] = zf
            zm[pl.ds(t * L, L)] = jnp.zeros((L,), jnp.int32)

        @pl.loop(0, CSE // L)
        def _(t):
            ones_buf[pl.ds(t * L, L)] = jnp.ones((L,), jnp.float32)

        @pl.loop(0, TPAD // NT // CSN)
        def _(i):
            pltpu.sync_copy(rows_a.at[pl.ds(0, CSN)],
                            acc_sh.at[pl.ds(s * (TPAD // NT) + i * CSN, CSN)])

        pltpu.sync_copy(zdeg, deg_sh.at[pl.ds(s * (TPAD // NT), TPAD // NT)])
        pltpu.sync_copy(zm, m_sh.at[pl.ds(s * (TPAD // NT), TPAD // NT)])

        @pl.when(s == 0)
        def _():
            pltpu.sync_copy(rows_a.at[pl.ds(0, 8)],
                            work.at[pl.ds(ci * ST + ZROW, 8)])

        plsc.subcore_barrier()

        @pl.when(s == 0)
        def _():
            @pl.loop(0, TPAD // (NB * CSE))
            def _(cb):
                pltpu.sync_copy(
                    e2t.at[pl.ds(ci * (TPAD // CSE) + cb * NB, NB)], sidx)

                @pl.loop(0, NB)
                def _(r):
                    jbase = cb * NB * CSE + r * CSE + 1

                    @pl.loop(0, CSE // L)
                    def _(q):
                        vals_buf[pl.ds(q * L, L)] = jbase + q * L + lane

                    pltpu.sync_copy(vals_buf, m_sh.at[sidx.at[r]])

        @pl.loop(0, nblk)
        def _(blk):
            pltpu.sync_copy(tile_dst.at[pl.ds(blk * NB, NB)], didx)

            @pl.loop(0, NB)
            def _(st):
                pltpu.sync_copy(ones_buf, deg_sh.at[didx.at[st]], add=True)

        plsc.subcore_barrier()

        for k in range(NCHN // NT):
            c = s * (NCHN // NT) + k

            @pl.when(c < NOUTCH)
            def _():
                base = c * CSN
                pltpu.sync_copy(deg_sh.at[pl.ds(base, CSN)], deg_buf)

                @pl.loop(0, CSN // L)
                def _(i):
                    d = deg_buf[pl.ds(i * L, L)] + 1.0
                    dis_t[pl.ds(k * CSN + i * L, L)] = _rsqrt16(d)

                pltpu.sync_copy(t2eg.at[pl.ds(ci * TPAD + base, CSN)],
                                idx_buf)
                pltpu.async_copy(H.at[idx_buf], rows_a.at[pl.ds(0, CSN)],
                                 sem_a).wait()

                @pl.loop(0, CSN)
                def _(r):
                    sv = plsc.load_gather(
                        dis_t, [jnp.full((L,), k * CSN + r, dtype=jnp.int32)])
                    for q in range(D // L):
                        sl = pl.ds(q * L, L)
                        rows_a[r, sl] = rows_a[r, sl] * sv

                pltpu.sync_copy(rows_a.at[pl.ds(0, CSN)],
                                work.at[pl.ds(ci * ST + base, CSN)])

        plsc.subcore_barrier()

        @pl.loop(0, nblk)
        def _(blk):
            pltpu.sync_copy(tile_src.at[pl.ds(blk * NB, NB)], sidx)
            pltpu.sync_copy(tile_dst.at[pl.ds(blk * NB, NB)], didx)

            @pl.loop(0, NB, step=2)
            def _(st):
                d_a = pltpu.async_copy(work.at[sidx.at[st]], rows_a, sem_a)
                d_b = pltpu.async_copy(work.at[sidx.at[st + 1]], rows_b,
                                       sem_b)
                d_a.wait()
                pltpu.sync_copy(rows_a, acc_sh.at[didx.at[st]], add=True)
                d_b.wait()
                pltpu.sync_copy(rows_b, acc_sh.at[didx.at[st + 1]], add=True)

        plsc.subcore_barrier()

        for k in range(NCHN // NT):
            c = s * (NCHN // NT) + k

            @pl.when(c < NOUTCH)
            def _():
                base = c * CSN
                pltpu.sync_copy(acc_sh.at[pl.ds(base, CSN)],
                                rows_a.at[pl.ds(0, CSN)])
                pltpu.sync_copy(work.at[pl.ds(ci * ST + base, CSN)],
                                rows_b.at[pl.ds(0, CSN)])

                @pl.loop(0, CSN)
                def _(r):
                    sv = plsc.load_gather(
                        dis_t, [jnp.full((L,), k * CSN + r, dtype=jnp.int32)])
                    for q in range(D // L):
                        sl = pl.ds(q * L, L)
                        rows_a[r, sl] = ((rows_a[r, sl] + rows_b[r, sl]) * sv
                                         + bias_buf[sl])

                pltpu.sync_copy(rows_a.at[pl.ds(0, CSN)],
                                work.at[pl.ds(ci * ST + base, CSN)])

        plsc.subcore_barrier()

        for k in range(NCHN // NT):
            c = s * (NCHN // NT) + k

            @pl.when(c < NOUTCH)
            def _():
                base = c * CSN
                pltpu.sync_copy(m_sh.at[pl.ds(base, CSN)], m_buf)
                off = ci * ST

                @pl.loop(0, CSN // L)
                def _(i):
                    mv = m_buf[pl.ds(i * L, L)]
                    idx_buf[pl.ds(i * L, L)] = (
                        jnp.where(mv >= 2, mv - 2, ZROW) + off)

                pltpu.async_copy(work.at[idx_buf], rows_a.at[pl.ds(0, CSN)],
                                 sem_a).wait()
                pltpu.sync_copy(x.at[pl.ds(ci * TPAD + base, CSN)],
                                rows_b.at[pl.ds(0, CSN)])

                @pl.loop(0, CSN)
                def _(r):
                    for q in range(D // L):
                        sl = pl.ds(q * L, L)
                        rows_a[r, sl] = rows_a[r, sl] + rows_b[r, sl]

                pltpu.sync_copy(rows_a.at[pl.ds(0, CSN)],
                                out.at[pl.ds(ci * T + base, CSN)])

    mesh = plsc.VectorSubcoreMesh(core_axis_name="c", subcore_axis_name="s")
    out_flat, _ = pl.kernel(
        body,
        out_type=[jax.ShapeDtypeStruct((B * T, D), jnp.float32),
                  jax.ShapeDtypeStruct((B * ST, D), jnp.float32)],
        mesh=mesh,
        compiler_params=pltpu.CompilerParams(needs_layout_passes=False),
        scratch_types=[
            pltpu.VMEM((NB, CSE), jnp.int32),
            pltpu.VMEM((NB, CSE), jnp.int32),
            pltpu.VMEM((CSE, D), jnp.float32),
            pltpu.VMEM((CSE, D), jnp.float32),
            pltpu.VMEM((CSN,), jnp.int32),
            pltpu.VMEM((CSN,), jnp.int32),
            pltpu.VMEM((CSN,), jnp.float32),
            pltpu.VMEM((8 * CSN,), jnp.float32),
            pltpu.VMEM((CSE,), jnp.float32),
            pltpu.VMEM((D,), jnp.float32),
            pltpu.VMEM((TPAD // NT,), jnp.float32),
            pltpu.VMEM((TPAD // NT,), jnp.int32),
            pltpu.VMEM((CSE,), jnp.int32),
            pltpu.VMEM_SHARED((TPAD, D), jnp.float32),
            pltpu.VMEM_SHARED((TPAD,), jnp.float32),
            pltpu.VMEM_SHARED((TPAD,), jnp.int32),
            pltpu.SemaphoreType.DMA,
            pltpu.SemaphoreType.DMA,
        ],
    )(H_flat, x_flat, src_g, dst_g, t2e_g, e2t_2d, b)
    return out_flat.reshape(B, T, D)


# speedup vs baseline: 12.2291x; 1.0124x over previous
import functools

import jax
import jax.numpy as jnp
from jax import lax
from jax.experimental import pallas as pl
from jax.experimental.pallas import tpu as pltpu
from jax.experimental.pallas import tpu_sc as plsc

L = 16
NT = 16
NC = 2
CSE = 128
CSN = 80
NB = 8
EB = 1024


def _rsqrt16(x):
    i = lax.bitcast_convert_type(x, jnp.int32)
    i = jnp.int32(0x5F3759DF) - (i >> 1)
    y = lax.bitcast_convert_type(i, jnp.float32)
    for _ in range(3):
        y = y * (1.5 - 0.5 * x * y * y)
    return y


def _matmul(x_flat, W):
    n, d = x_flat.shape
    blk = 1024

    def body(x_ref, w_ref, o_ref):
        o_ref[...] = jnp.dot(x_ref[...], w_ref[...],
                             preferred_element_type=jnp.float32)

    return pl.pallas_call(
        body,
        grid=(n // blk,),
        in_specs=[pl.BlockSpec((blk, d), lambda i: (i, 0)),
                  pl.BlockSpec((d, d), lambda i: (0, 0))],
        out_specs=pl.BlockSpec((blk, d), lambda i: (i, 0)),
        out_shape=jax.ShapeDtypeStruct((n, d), jnp.float32),
    )(x_flat, W)


def kernel(token_embeddings, tokens2edges, edge_index, edges2tokens, W, b):
    B, T, D = token_embeddings.shape
    E = edge_index.shape[2]
    TPAD = ((T + NT * CSN - 1) // (NT * CSN)) * (NT * CSN)
    ST = TPAD + 8
    ZROW = TPAD
    NCHN = TPAD // CSN
    NOUTCH = T // CSN
    steps = -(-E // (NT * CSE))
    steps += -steps % NB
    nblk = steps // NB
    pad_e = NT * steps * CSE - E

    x_flat = jnp.pad(token_embeddings, ((0, 0), (0, TPAD - T), (0, 0))
                     ).reshape(B * TPAD, D)
    boffs_st = (jnp.arange(B, dtype=jnp.int32) * ST)[:, None]
    boffs_tp = (jnp.arange(B, dtype=jnp.int32) * TPAD)[:, None]
    src_g = (jnp.pad(edge_index[:, 0, :], ((0, 0), (0, pad_e))) + boffs_st
             ).reshape(B * NT, steps, CSE)
    dst_g = jnp.pad(edge_index[:, 1, :], ((0, 0), (0, pad_e)),
                    constant_values=T).reshape(B * NT, steps, CSE)
    t2e_g = (jnp.pad(tokens2edges, ((0, 0), (0, TPAD - T))) + boffs_tp
             ).reshape(B * TPAD)
    e2t_2d = jnp.pad(edges2tokens, ((0, 0), (0, TPAD - T)),
                     constant_values=T).reshape(B * TPAD // CSE, CSE)

    H_flat = _matmul(x_flat, W)

    def body(H, x, srcg, dstg, t2eg, e2t, bvec, out, work,
             sidx, didx, rows_a, rows_b, idx_buf, m_buf, deg_buf,
             dis_t, ones_buf, bias_buf, zdeg, zm, vals_buf,
             acc_sh, deg_sh, m_sh, sem_a, sem_b, sem_c, sem_d):
        ci = lax.axis_index("c")
        s = lax.axis_index("s")
        lane = lax.iota(jnp.int32, L)
        tile_src = srcg.at[ci * NT + s]
        tile_dst = dstg.at[ci * NT + s]

        pltpu.sync_copy(bvec, bias_buf)
        zf = jnp.zeros((L,), jnp.float32)

        @pl.loop(0, CSE * D // L)
        def _(t):
            rows_a[t >> 3, pl.ds((t & 7) * L, L)] = zf

        @pl.loop(0, TPAD // NT // L)
        def _(t):
            zdeg[pl.ds(t * L, L)] = zf
            zm[pl.ds(t * L, L)] = jnp.zeros((L,), jnp.int32)

        @pl.loop(0, CSE // L)
        def _(t):
            ones_buf[pl.ds(t * L, L)] = jnp.ones((L,), jnp.float32)

        @pl.loop(0, TPAD // NT // CSN)
        def _(i):
            pltpu.sync_copy(rows_a.at[pl.ds(0, CSN)],
                            acc_sh.at[pl.ds(s * (TPAD // NT) + i * CSN, CSN)])

        pltpu.sync_copy(zdeg, deg_sh.at[pl.ds(s * (TPAD // NT), TPAD // NT)])
        pltpu.sync_copy(zm, m_sh.at[pl.ds(s * (TPAD // NT), TPAD // NT)])

        @pl.when(s == 0)
        def _():
            pltpu.sync_copy(rows_a.at[pl.ds(0, 8)],
                            work.at[pl.ds(ci * ST + ZROW, 8)])

        plsc.subcore_barrier()

        @pl.when(s == 0)
        def _():
            @pl.loop(0, TPAD // (NB * CSE))
            def _(cb):
                pltpu.sync_copy(
                    e2t.at[pl.ds(ci * (TPAD // CSE) + cb * NB, NB)], sidx)

                @pl.loop(0, NB)
                def _(r):
                    jbase = cb * NB * CSE + r * CSE + 1

                    @pl.loop(0, CSE // L)
                    def _(q):
                        vals_buf[pl.ds(q * L, L)] = jbase + q * L + lane

                    pltpu.sync_copy(vals_buf, m_sh.at[sidx.at[r]])

        @pl.loop(0, nblk)
        def _(blk):
            pltpu.sync_copy(tile_dst.at[pl.ds(blk * NB, NB)], didx)
            ds_ = [pltpu.async_copy(ones_buf, deg_sh.at[didx.at[st]],
                                    sem_c, add=True) for st in range(NB)]
            for d in ds_:
                d.wait()

        plsc.subcore_barrier()

        for k in range(NCHN // NT):
            c = s * (NCHN // NT) + k

            @pl.when(c < NOUTCH)
            def _():
                base = c * CSN
                pltpu.sync_copy(deg_sh.at[pl.ds(base, CSN)], deg_buf)

                @pl.loop(0, CSN // L)
                def _(i):
                    d = deg_buf[pl.ds(i * L, L)] + 1.0
                    dis_t[pl.ds(k * CSN + i * L, L)] = _rsqrt16(d)

                pltpu.sync_copy(t2eg.at[pl.ds(ci * TPAD + base, CSN)],
                                idx_buf)
                pltpu.async_copy(H.at[idx_buf], rows_a.at[pl.ds(0, CSN)],
                                 sem_a).wait()

                @pl.loop(0, CSN)
                def _(r):
                    sv = plsc.load_gather(
                        dis_t, [jnp.full((L,), k * CSN + r, dtype=jnp.int32)])
                    for q in range(D // L):
                        sl = pl.ds(q * L, L)
                        rows_a[r, sl] = rows_a[r, sl] * sv

                pltpu.sync_copy(rows_a.at[pl.ds(0, CSN)],
                                work.at[pl.ds(ci * ST + base, CSN)])

        plsc.subcore_barrier()

        @pl.loop(0, nblk)
        def _(blk):
            pltpu.sync_copy(tile_src.at[pl.ds(blk * NB, NB)], sidx)
            pltpu.sync_copy(tile_dst.at[pl.ds(blk * NB, NB)], didx)

            @pl.loop(0, NB, step=2)
            def _(st):
                d_a = pltpu.async_copy(work.at[sidx.at[st]], rows_a, sem_a)
                d_b = pltpu.async_copy(work.at[sidx.at[st + 1]], rows_b,
                                       sem_b)
                d_a.wait()
                s_a = pltpu.async_copy(rows_a, acc_sh.at[didx.at[st]],
                                       sem_c, add=True)
                d_b.wait()
                s_b = pltpu.async_copy(rows_b, acc_sh.at[didx.at[st + 1]],
                                       sem_d, add=True)
                s_a.wait()
                s_b.wait()

        plsc.subcore_barrier()

        for k in range(NCHN // NT):
            c = s * (NCHN // NT) + k

            @pl.when(c < NOUTCH)
            def _():
                base = c * CSN
                pltpu.sync_copy(acc_sh.at[pl.ds(base, CSN)],
                                rows_a.at[pl.ds(0, CSN)])
                pltpu.sync_copy(work.at[pl.ds(ci * ST + base, CSN)],
                                rows_b.at[pl.ds(0, CSN)])

                @pl.loop(0, CSN)
                def _(r):
                    sv = plsc.load_gather(
                        dis_t, [jnp.full((L,), k * CSN + r, dtype=jnp.int32)])
                    for q in range(D // L):
                        sl = pl.ds(q * L, L)
                        rows_a[r, sl] = ((rows_a[r, sl] + rows_b[r, sl]) * sv
                                         + bias_buf[sl])

                pltpu.sync_copy(rows_a.at[pl.ds(0, CSN)],
                                work.at[pl.ds(ci * ST + base, CSN)])

        plsc.subcore_barrier()

        for k in range(NCHN // NT):
            c = s * (NCHN // NT) + k

            @pl.when(c < NOUTCH)
            def _():
                base = c * CSN
                pltpu.sync_copy(m_sh.at[pl.ds(base, CSN)], m_buf)
                off = ci * ST

                @pl.loop(0, CSN // L)
                def _(i):
                    mv = m_buf[pl.ds(i * L, L)]
                    idx_buf[pl.ds(i * L, L)] = (
                        jnp.where(mv >= 2, mv - 2, ZROW) + off)

                pltpu.async_copy(work.at[idx_buf], rows_a.at[pl.ds(0, CSN)],
                                 sem_a).wait()
                pltpu.sync_copy(x.at[pl.ds(ci * TPAD + base, CSN)],
                                rows_b.at[pl.ds(0, CSN)])

                @pl.loop(0, CSN)
                def _(r):
                    for q in range(D // L):
                        sl = pl.ds(q * L, L)
                        rows_a[r, sl] = rows_a[r, sl] + rows_b[r, sl]

                pltpu.sync_copy(rows_a.at[pl.ds(0, CSN)],
                                out.at[pl.ds(ci * T + base, CSN)])

    mesh = plsc.VectorSubcoreMesh(core_axis_name="c", subcore_axis_name="s")
    out_flat, _ = pl.kernel(
        body,
        out_type=[jax.ShapeDtypeStruct((B * T, D), jnp.float32),
                  jax.ShapeDtypeStruct((B * ST, D), jnp.float32)],
        mesh=mesh,
        compiler_params=pltpu.CompilerParams(needs_layout_passes=False),
        scratch_types=[
            pltpu.VMEM((NB, CSE), jnp.int32),
            pltpu.VMEM((NB, CSE), jnp.int32),
            pltpu.VMEM((CSE, D), jnp.float32),
            pltpu.VMEM((CSE, D), jnp.float32),
            pltpu.VMEM((CSN,), jnp.int32),
            pltpu.VMEM((CSN,), jnp.int32),
            pltpu.VMEM((CSN,), jnp.float32),
            pltpu.VMEM((8 * CSN,), jnp.float32),
            pltpu.VMEM((CSE,), jnp.float32),
            pltpu.VMEM((D,), jnp.float32),
            pltpu.VMEM((TPAD // NT,), jnp.float32),
            pltpu.VMEM((TPAD // NT,), jnp.int32),
            pltpu.VMEM((CSE,), jnp.int32),
            pltpu.VMEM_SHARED((TPAD, D), jnp.float32),
            pltpu.VMEM_SHARED((TPAD,), jnp.float32),
            pltpu.VMEM_SHARED((TPAD,), jnp.int32),
            pltpu.SemaphoreType.DMA,
            pltpu.SemaphoreType.DMA,
            pltpu.SemaphoreType.DMA,
            pltpu.SemaphoreType.DMA,
        ],
    )(H_flat, x_flat, src_g, dst_g, t2e_g, e2t_2d, b)
    return out_flat.reshape(B, T, D)


# speedup vs baseline: 12.2486x; 1.0016x over previous
import functools

import jax
import jax.numpy as jnp
from jax import lax
from jax.experimental import pallas as pl
from jax.experimental.pallas import tpu as pltpu
from jax.experimental.pallas import tpu_sc as plsc

L = 16
NT = 16
NC = 2
CSE = 128
CSN = 80
NB = 8
EB = 1024


def _rsqrt16(x):
    i = lax.bitcast_convert_type(x, jnp.int32)
    i = jnp.int32(0x5F3759DF) - (i >> 1)
    y = lax.bitcast_convert_type(i, jnp.float32)
    for _ in range(3):
        y = y * (1.5 - 0.5 * x * y * y)
    return y


def _matmul(x_flat, W):
    n, d = x_flat.shape
    blk = 1024

    def body(x_ref, w_ref, o_ref):
        o_ref[...] = jnp.dot(x_ref[...], w_ref[...],
                             preferred_element_type=jnp.float32)

    return pl.pallas_call(
        body,
        grid=(n // blk,),
        in_specs=[pl.BlockSpec((blk, d), lambda i: (i, 0)),
                  pl.BlockSpec((d, d), lambda i: (0, 0))],
        out_specs=pl.BlockSpec((blk, d), lambda i: (i, 0)),
        out_shape=jax.ShapeDtypeStruct((n, d), jnp.float32),
    )(x_flat, W)


def kernel(token_embeddings, tokens2edges, edge_index, edges2tokens, W, b):
    B, T, D = token_embeddings.shape
    E = edge_index.shape[2]
    TPAD = ((T + NT * CSN - 1) // (NT * CSN)) * (NT * CSN)
    ST = TPAD + 8
    ZROW = TPAD
    NCHN = TPAD // CSN
    NOUTCH = T // CSN
    steps = -(-E // (NT * CSE))
    steps += -steps % NB
    nblk = steps // NB
    pad_e = NT * steps * CSE - E

    x_flat = jnp.pad(token_embeddings, ((0, 0), (0, TPAD - T), (0, 0))
                     ).reshape(B * TPAD, D)
    boffs_st = (jnp.arange(B, dtype=jnp.int32) * ST)[:, None]
    boffs_tp = (jnp.arange(B, dtype=jnp.int32) * TPAD)[:, None]
    src_g = (jnp.pad(edge_index[:, 0, :], ((0, 0), (0, pad_e))) + boffs_st
             ).reshape(B * NT, steps, CSE)
    dst_g = jnp.pad(edge_index[:, 1, :], ((0, 0), (0, pad_e)),
                    constant_values=T).reshape(B * NT, steps, CSE)
    t2e_g = (jnp.pad(tokens2edges, ((0, 0), (0, TPAD - T))) + boffs_tp
             ).reshape(B * TPAD)
    e2t_2d = jnp.pad(edges2tokens, ((0, 0), (0, TPAD - T)),
                     constant_values=T).reshape(B * TPAD // CSE, CSE)

    H_flat = _matmul(x_flat, W)

    def body(H, x, srcg, dstg, t2eg, e2t, bvec, out, work,
             sidx, didx, rows_a, rows_b, idx_buf, m_buf, deg_buf,
             dis_t, ones_buf, bias_buf, zdeg, zm, vals_buf,
             acc_sh, deg_sh, m_sh, sem_a, sem_b, sem_c, sem_d):
        ci = lax.axis_index("c")
        s = lax.axis_index("s")
        lane = lax.iota(jnp.int32, L)
        tile_src = srcg.at[ci * NT + s]
        tile_dst = dstg.at[ci * NT + s]

        pltpu.sync_copy(bvec, bias_buf)
        zf = jnp.zeros((L,), jnp.float32)

        @pl.loop(0, CSE * D // L)
        def _(t):
            rows_a[t >> 3, pl.ds((t & 7) * L, L)] = zf

        @pl.loop(0, TPAD // NT // L)
        def _(t):
            zdeg[pl.ds(t * L, L)] = zf
            zm[pl.ds(t * L, L)] = jnp.zeros((L,), jnp.int32)

        @pl.loop(0, CSE // L)
        def _(t):
            ones_buf[pl.ds(t * L, L)] = jnp.ones((L,), jnp.float32)

        @pl.loop(0, TPAD // NT // CSN)
        def _(i):
            pltpu.sync_copy(rows_a.at[pl.ds(0, CSN)],
                            acc_sh.at[pl.ds(s * (TPAD // NT) + i * CSN, CSN)])

        pltpu.sync_copy(zdeg, deg_sh.at[pl.ds(s * (TPAD // NT), TPAD // NT)])
        pltpu.sync_copy(zm, m_sh.at[pl.ds(s * (TPAD // NT), TPAD // NT)])

        @pl.when(s == 0)
        def _():
            pltpu.sync_copy(rows_a.at[pl.ds(0, 8)],
                            work.at[pl.ds(ci * ST + ZROW, 8)])

        plsc.subcore_barrier()

        @pl.when(s == 0)
        def _():
            @pl.loop(0, TPAD // (NB * CSE))
            def _(cb):
                pltpu.sync_copy(
                    e2t.at[pl.ds(ci * (TPAD // CSE) + cb * NB, NB)], sidx)

                @pl.loop(0, NB)
                def _(r):
                    jbase = cb * NB * CSE + r * CSE + 1

                    @pl.loop(0, CSE // L)
                    def _(q):
                        vals_buf[r, pl.ds(q * L, L)] = jbase + q * L + lane

                ms_ = [pltpu.async_copy(vals_buf.at[r], m_sh.at[sidx.at[r]],
                                        sem_d) for r in range(NB)]
                for d in ms_:
                    d.wait()

        def _deg_blocks(dref, lo, hi):
            @pl.loop(lo, hi)
            def _(blk):
                pltpu.sync_copy(dref.at[pl.ds(blk * NB, NB)], didx)
                ds_ = [pltpu.async_copy(ones_buf, deg_sh.at[didx.at[st]],
                                        sem_c, add=True) for st in range(NB)]
                for d in ds_:
                    d.wait()

        @pl.when(s == 0)
        def _():
            _deg_blocks(tile_dst, 0, nblk // 2)

        @pl.when(s != 0)
        def _():
            _deg_blocks(tile_dst, 0, nblk)

        @pl.when(s == NT - 1)
        def _():
            _deg_blocks(dstg.at[ci * NT], nblk // 2, nblk)

        plsc.subcore_barrier()

        for k in range(NCHN // NT):
            c = s * (NCHN // NT) + k

            @pl.when(c < NOUTCH)
            def _():
                base = c * CSN
                pltpu.sync_copy(deg_sh.at[pl.ds(base, CSN)], deg_buf)

                @pl.loop(0, CSN // L)
                def _(i):
                    d = deg_buf[pl.ds(i * L, L)] + 1.0
                    dis_t[pl.ds(k * CSN + i * L, L)] = _rsqrt16(d)

                pltpu.sync_copy(t2eg.at[pl.ds(ci * TPAD + base, CSN)],
                                idx_buf)
                pltpu.async_copy(H.at[idx_buf], rows_a.at[pl.ds(0, CSN)],
                                 sem_a).wait()

                @pl.loop(0, CSN)
                def _(r):
                    sv = plsc.load_gather(
                        dis_t, [jnp.full((L,), k * CSN + r, dtype=jnp.int32)])
                    for q in range(D // L):
                        sl = pl.ds(q * L, L)
                        rows_a[r, sl] = rows_a[r, sl] * sv

                pltpu.sync_copy(rows_a.at[pl.ds(0, CSN)],
                                work.at[pl.ds(ci * ST + base, CSN)])

        plsc.subcore_barrier()

        @pl.loop(0, nblk)
        def _(blk):
            pltpu.sync_copy(tile_src.at[pl.ds(blk * NB, NB)], sidx)
            pltpu.sync_copy(tile_dst.at[pl.ds(blk * NB, NB)], didx)

            @pl.loop(0, NB, step=2)
            def _(st):
                d_a = pltpu.async_copy(work.at[sidx.at[st]], rows_a, sem_a)
                d_b = pltpu.async_copy(work.at[sidx.at[st + 1]], rows_b,
                                       sem_b)
                d_a.wait()
                s_a = pltpu.async_copy(rows_a, acc_sh.at[didx.at[st]],
                                       sem_c, add=True)
                d_b.wait()
                s_b = pltpu.async_copy(rows_b, acc_sh.at[didx.at[st + 1]],
                                       sem_d, add=True)
                s_a.wait()
                s_b.wait()

        plsc.subcore_barrier()

        for k in range(NCHN // NT):
            c = s * (NCHN // NT) + k

            @pl.when(c < NOUTCH)
            def _():
                base = c * CSN
                pltpu.sync_copy(acc_sh.at[pl.ds(base, CSN)],
                                rows_a.at[pl.ds(0, CSN)])
                pltpu.sync_copy(work.at[pl.ds(ci * ST + base, CSN)],
                                rows_b.at[pl.ds(0, CSN)])

                @pl.loop(0, CSN)
                def _(r):
                    sv = plsc.load_gather(
                        dis_t, [jnp.full((L,), k * CSN + r, dtype=jnp.int32)])
                    for q in range(D // L):
                        sl = pl.ds(q * L, L)
                        rows_a[r, sl] = ((rows_a[r, sl] + rows_b[r, sl]) * sv
                                         + bias_buf[sl])

                pltpu.sync_copy(rows_a.at[pl.ds(0, CSN)],
                                work.at[pl.ds(ci * ST + base, CSN)])

        plsc.subcore_barrier()

        for k in range(NCHN // NT):
            c = s * (NCHN // NT) + k

            @pl.when(c < NOUTCH)
            def _():
                base = c * CSN
                pltpu.sync_copy(m_sh.at[pl.ds(base, CSN)], m_buf)
                off = ci * ST

                @pl.loop(0, CSN // L)
                def _(i):
                    mv = m_buf[pl.ds(i * L, L)]
                    idx_buf[pl.ds(i * L, L)] = (
                        jnp.where(mv >= 2, mv - 2, ZROW) + off)

                pltpu.async_copy(work.at[idx_buf], rows_a.at[pl.ds(0, CSN)],
                                 sem_a).wait()
                pltpu.sync_copy(x.at[pl.ds(ci * TPAD + base, CSN)],
                                rows_b.at[pl.ds(0, CSN)])

                @pl.loop(0, CSN)
                def _(r):
                    for q in range(D // L):
                        sl = pl.ds(q * L, L)
                        rows_a[r, sl] = rows_a[r, sl] + rows_b[r, sl]

                pltpu.sync_copy(rows_a.at[pl.ds(0, CSN)],
                                out.at[pl.ds(ci * T + base, CSN)])

    mesh = plsc.VectorSubcoreMesh(core_axis_name="c", subcore_axis_name="s")
    out_flat, _ = pl.kernel(
        body,
        out_type=[jax.ShapeDtypeStruct((B * T, D), jnp.float32),
                  jax.ShapeDtypeStruct((B * ST, D), jnp.float32)],
        mesh=mesh,
        compiler_params=pltpu.CompilerParams(needs_layout_passes=False),
        scratch_types=[
            pltpu.VMEM((NB, CSE), jnp.int32),
            pltpu.VMEM((NB, CSE), jnp.int32),
            pltpu.VMEM((CSE, D), jnp.float32),
            pltpu.VMEM((CSE, D), jnp.float32),
            pltpu.VMEM((CSN,), jnp.int32),
            pltpu.VMEM((CSN,), jnp.int32),
            pltpu.VMEM((CSN,), jnp.float32),
            pltpu.VMEM((8 * CSN,), jnp.float32),
            pltpu.VMEM((CSE,), jnp.float32),
            pltpu.VMEM((D,), jnp.float32),
            pltpu.VMEM((TPAD // NT,), jnp.float32),
            pltpu.VMEM((TPAD // NT,), jnp.int32),
            pltpu.VMEM((NB, CSE), jnp.int32),
            pltpu.VMEM_SHARED((TPAD, D), jnp.float32),
            pltpu.VMEM_SHARED((TPAD,), jnp.float32),
            pltpu.VMEM_SHARED((TPAD,), jnp.int32),
            pltpu.SemaphoreType.DMA,
            pltpu.SemaphoreType.DMA,
            pltpu.SemaphoreType.DMA,
            pltpu.SemaphoreType.DMA,
        ],
    )(H_flat, x_flat, src_g, dst_g, t2e_g, e2t_2d, b)
    return out_flat.reshape(B, T, D)
